# baseline (speedup 1.0000x reference)
import jax
import jax.numpy as jnp
from jax.experimental import pallas as pl
from jax.experimental.pallas import tpu as pltpu

B, S, D = 8, 512, 768
C = 5
H1 = 768
H2 = 384
CP = 128

_C1 = 0.7978845608028654
_C2 = _C1 * 0.044715


def _gelu(x):
    t = jnp.tanh(x * (_C1 + _C2 * (x * x)))
    return x * (0.5 * t + 0.5)


SH = S // 2


def _mlp_body(nseg_ref, q_ref, seg_ref, tis_ref, w1q_ref, w1s_ref, b1_ref,
              w2_ref, b2_ref, w3_ref, b3_ref, probs_ref, tinst_ref, qh_s):
    b = pl.program_id(0)
    s = pl.program_id(1)

    @pl.when((b == 0) & (s == 0))
    def _init():
        qh_s[...] = jnp.dot(q_ref[...], w1q_ref[...],
                            preferred_element_type=jnp.float32)

    n = nseg_ref[b]
    row = jax.lax.broadcasted_iota(jnp.int32, (SH, 1), 0) + s * SH
    x = jnp.where(row < n, seg_ref[0], 0.0)

    h1 = _gelu(jnp.dot(x, w1s_ref[...], preferred_element_type=jnp.float32)
               + qh_s[pl.ds(b, 1)] + b1_ref[...])
    h2 = _gelu(jnp.dot(h1, w2_ref[...], preferred_element_type=jnp.float32)
               + b2_ref[...])
    logits = jnp.dot(h2, w3_ref[...], preferred_element_type=jnp.float32)
    logits = logits + b3_ref[...]
    m = jnp.max(logits, axis=-1, keepdims=True)
    e = jnp.exp(logits - m)
    probs = e / jnp.sum(e, axis=-1, keepdims=True)
    probs_ref[0] = probs[:, :C]

    @pl.when(s == 0)
    def _tinst():
        col = jax.lax.broadcasted_iota(jnp.int32, (1, S), 1)
        tinst_ref[pl.ds(b, 1)] = jnp.where(col < n, tis_ref[pl.ds(b, 1)],
                                           1.0)


def kernel(questions_embedding, context_segments_embedding,
           num_context_segments, target_agg_score, target_instance_score,
           W1, b1, W2, b2, W3, b3):
    b1_2d = b1.reshape(1, H1)
    b2_2d = b2.reshape(1, H2)
    w3p = jnp.pad(W3, ((0, 0), (0, CP - C)))
    b3p = jnp.concatenate([b3, jnp.full((CP - C,), -1e30, jnp.float32)])
    b3p = b3p.reshape(1, CP)

    grid_spec = pltpu.PrefetchScalarGridSpec(
        num_scalar_prefetch=1,
        grid=(B, 2),
        in_specs=[
            pl.BlockSpec((B, D), lambda b, s, n: (0, 0)),
            pl.BlockSpec((1, SH, D), lambda b, s, n: (b, s, 0)),
            pl.BlockSpec((B, S), lambda b, s, n: (0, 0)),
            pl.BlockSpec((D, H1), lambda b, s, n: (0, 0)),
            pl.BlockSpec((D, H1), lambda b, s, n: (1, 0)),
            pl.BlockSpec((1, H1), lambda b, s, n: (0, 0)),
            pl.BlockSpec((H1, H2), lambda b, s, n: (0, 0)),
            pl.BlockSpec((1, H2), lambda b, s, n: (0, 0)),
            pl.BlockSpec((H2, CP), lambda b, s, n: (0, 0)),
            pl.BlockSpec((1, CP), lambda b, s, n: (0, 0)),
        ],
        out_specs=[
            pl.BlockSpec((1, SH, C), lambda b, s, n: (b, s, 0)),
            pl.BlockSpec((B, S), lambda b, s, n: (0, 0)),
        ],
        scratch_shapes=[
            pltpu.VMEM((B, H1), jnp.float32),
        ],
    )

    probs, tinst = pl.pallas_call(
        _mlp_body,
        grid_spec=grid_spec,
        out_shape=[
            jax.ShapeDtypeStruct((B, S, C), jnp.float32),
            jax.ShapeDtypeStruct((B, S), jnp.float32),
        ],
    )(num_context_segments, questions_embedding, context_segments_embedding,
      target_instance_score, W1, W1, b1_2d, W2, b2_2d, w3p, b3p)

    return (target_agg_score, tinst, probs, num_context_segments)


# speedup vs baseline: 1.1994x; 1.1994x over previous
import jax
import jax.numpy as jnp
from jax.experimental import pallas as pl
from jax.experimental.pallas import tpu as pltpu

B, S, D = 8, 512, 768
C = 5
H1 = 768
H2 = 384
CP = 128

_C1 = 0.7978845608028654
_C2 = _C1 * 0.044715


def _gelu(x):
    t = jnp.tanh(x * (_C1 + _C2 * (x * x)))
    return x * (0.5 * t + 0.5)


def _mlp_body(nseg_ref, q_ref, seg_ref, tis_ref, w1q_ref, w1s_ref, b1_ref,
              w2_ref, b2_ref, w3_ref, b3_ref, probs_ref, tinst_ref, qh_s):
    b = pl.program_id(0)

    @pl.when(b == 0)
    def _init():
        qh_s[...] = jnp.dot(q_ref[...], w1q_ref[...],
                            preferred_element_type=jnp.float32)

    n = nseg_ref[b]
    row = jax.lax.broadcasted_iota(jnp.int32, (S, 1), 0)
    x = jnp.where(row < n, seg_ref[0], 0.0)

    h1 = _gelu(jnp.dot(x, w1s_ref[...], preferred_element_type=jnp.float32)
               + qh_s[pl.ds(b, 1)] + b1_ref[...])
    h2 = _gelu(jnp.dot(h1, w2_ref[...], preferred_element_type=jnp.float32)
               + b2_ref[...])
    logits = jnp.dot(h2, w3_ref[...], preferred_element_type=jnp.float32)
    logits = logits + b3_ref[...]
    m = jnp.max(logits, axis=-1, keepdims=True)
    e = jnp.exp(logits - m)
    probs = e / jnp.sum(e, axis=-1, keepdims=True)
    probs_ref[0] = probs[:, :C]

    col = jax.lax.broadcasted_iota(jnp.int32, (1, S), 1)
    tinst_ref[pl.ds(b, 1)] = jnp.where(col < n, tis_ref[pl.ds(b, 1)], 1.0)


def kernel(questions_embedding, context_segments_embedding,
           num_context_segments, target_agg_score, target_instance_score,
           W1, b1, W2, b2, W3, b3):
    b1_2d = b1.reshape(1, H1)
    b2_2d = b2.reshape(1, H2)
    w3p = jnp.pad(W3, ((0, 0), (0, CP - C)))
    b3p = jnp.concatenate([b3, jnp.full((CP - C,), -1e30, jnp.float32)])
    b3p = b3p.reshape(1, CP)

    grid_spec = pltpu.PrefetchScalarGridSpec(
        num_scalar_prefetch=1,
        grid=(B,),
        in_specs=[
            pl.BlockSpec((B, D), lambda b, n: (0, 0)),
            pl.BlockSpec((1, S, D), lambda b, n: (b, 0, 0)),
            pl.BlockSpec((B, S), lambda b, n: (0, 0)),
            pl.BlockSpec((D, H1), lambda b, n: (0, 0)),
            pl.BlockSpec((D, H1), lambda b, n: (1, 0)),
            pl.BlockSpec((1, H1), lambda b, n: (0, 0)),
            pl.BlockSpec((H1, H2), lambda b, n: (0, 0)),
            pl.BlockSpec((1, H2), lambda b, n: (0, 0)),
            pl.BlockSpec((H2, CP), lambda b, n: (0, 0)),
            pl.BlockSpec((1, CP), lambda b, n: (0, 0)),
        ],
        out_specs=[
            pl.BlockSpec((1, S, C), lambda b, n: (b, 0, 0)),
            pl.BlockSpec((B, S), lambda b, n: (0, 0)),
        ],
        scratch_shapes=[
            pltpu.VMEM((B, H1), jnp.float32),
        ],
    )

    probs, tinst = pl.pallas_call(
        _mlp_body,
        grid_spec=grid_spec,
        out_shape=[
            jax.ShapeDtypeStruct((B, S, C), jnp.float32),
            jax.ShapeDtypeStruct((B, S), jnp.float32),
        ],
    )(num_context_segments, questions_embedding, context_segments_embedding,
      target_instance_score, W1, W1, b1_2d, W2, b2_2d, w3p, b3p)

    return (target_agg_score, tinst, probs, num_context_segments)
